# baseline (speedup 1.0000x reference)
import functools

import jax
import jax.numpy as jnp
from jax import lax
from jax.experimental import pallas as pl
from jax.experimental.pallas import tpu as pltpu
from jax.experimental.pallas import tpu_sc as plsc

B = 16384
VOCAB = 1000000
DIM = 32
NBUCKETS = 1000
OUT_COLS = 2 * DIM + 1
OUT_PAD = 72

NC = 2
NS = 16
L = 16
NW = NC * NS
BPW = B // NW
NVEC = BPW // L
BKT_PAD = 1024
CONST_PAD = 128
K = 4
NGRP = BPW // K

NTILE = VOCAB // 128
TAIL_BASE = NTILE * 128
TAIL_N = VOCAB - TAIL_BASE
TS_ROWS_PAD = 1024


def _sc_body(uid_hbm, ts_hbm, tabT_hbm, tail_hbm, tts_hbm, bkt_hbm,
             consts_hbm, out_hbm, uid_v, ts_v, tidx_v, bkt_v, consts_v,
             tail_v, tts_v, blk0, blk1, blk2, blk3, blk4, blk5, blk6, blk7,
             stage_v, sem_a, sem_b):
    wid = lax.axis_index("s") * NC + lax.axis_index("c")
    base = wid * BPW
    slot_a = [blk0, blk1, blk2, blk3]
    slot_b = [blk4, blk5, blk6, blk7]
    lanes = lax.iota(jnp.int32, L)

    pltpu.sync_copy(uid_hbm.at[pl.ds(base, BPW)], uid_v)
    pltpu.sync_copy(ts_hbm.at[pl.ds(base, BPW)], ts_v)
    pltpu.sync_copy(bkt_hbm, bkt_v)
    pltpu.sync_copy(consts_hbm, consts_v)
    pltpu.sync_copy(tail_hbm, tail_v)
    pltpu.sync_copy(tts_hbm, tts_v)

    inv_step = consts_v[pl.ds(0, L)]
    mean = consts_v[pl.ds(L, L)]
    denom = consts_v[pl.ds(2 * L, L)]

    def uscalar(j):
        voff = pl.multiple_of((j >> 4) * L, L)
        vec = uid_v[pl.ds(voff, L)]
        return jnp.sum(jnp.where(lanes == (j & (L - 1)), vec, 0))

    def fire_group(slots, sem, gi, hb):
        us = []
        for k in range(K):
            j = jnp.minimum(gi * K + k, BPW - 1) + hb
            u = uscalar(j)
            utile = jnp.minimum(u >> 7, NTILE - 1)
            off = pl.multiple_of(utile * 128, 128)
            pltpu.async_copy(tabT_hbm.at[:, pl.ds(off, 128)], slots[k], sem)
            us.append(u)
        return tuple(us)

    def drain(slots, sem):
        for k in range(K):
            pltpu.make_async_copy(tabT_hbm.at[:, pl.ds(0, 128)],
                                  slots[k], sem).wait()

    def extract_group(slots, us, gi):
        for k in range(K):
            j = jnp.minimum(gi * K + k, BPW - 1)
            u = us[k]
            ucol = jnp.full((L,), u & 127, jnp.int32)
            istail = u >= TAIL_BASE
            uloc = jnp.clip(u - TAIL_BASE, 0, TAIL_N - 1)
            jcol = jnp.full((L,), j, jnp.int32)
            for c in range(DIM // L):
                gn = plsc.load_gather(slots[k], [c * L + lanes, ucol])
                toff = pl.multiple_of(uloc * DIM + c * L, L)
                gt = tail_v[pl.ds(toff, L)]
                plsc.store_scatter(stage_v, [c * L + lanes, jcol],
                                   jnp.where(istail, gt, gn))

    for i in range(NVEC):
        t = ts_v[pl.ds(i * L, L)]
        g = jnp.clip((t * inv_step).astype(jnp.int32), 0, NBUCKETS - 2)
        blo = plsc.load_gather(bkt_v, [g])
        bhi = plsc.load_gather(bkt_v, [g + 1])
        idx = jnp.where(t < blo, g, jnp.where(t >= bhi, g + 2, g + 1))
        tidx_v[pl.ds(i * L, L)] = idx
        stage_v[2 * DIM, pl.ds(i * L, L)] = (t - mean) / denom

    carry = fire_group(slot_a, sem_a, 0, 0)

    def two_groups(t2, carry):
        ua = carry
        ub = fire_group(slot_b, sem_b, 2 * t2 + 1, 0)
        drain(slot_a, sem_a)
        extract_group(slot_a, ua, 2 * t2)
        un = fire_group(slot_a, sem_a, 2 * t2 + 2, 0)
        drain(slot_b, sem_b)
        extract_group(slot_b, ub, 2 * t2 + 1)
        return un

    carry = lax.fori_loop(0, NGRP // 2, two_groups, carry)
    drain(slot_a, sem_a)

    for i in range(NVEC):
        tidx = tidx_v[pl.ds(i * L, L)]
        for d in range(DIM):
            vals = plsc.load_gather(tts_v, [tidx * DIM + d])
            stage_v[DIM + d, pl.ds(i * L, L)] = vals

    pltpu.sync_copy(stage_v, out_hbm.at[:, pl.ds(base, BPW)])


@jax.jit
def _run(user_id, timestamp, tabT, tail, tts, buckets_pad, consts):
    mesh = plsc.VectorSubcoreMesh(core_axis_name="c", subcore_axis_name="s")
    f = functools.partial(
        pl.kernel,
        mesh=mesh,
        compiler_params=pltpu.CompilerParams(needs_layout_passes=False),
        out_type=jax.ShapeDtypeStruct((OUT_PAD, B), jnp.float32),
        scratch_types=[
            pltpu.VMEM((BPW,), jnp.int32),
            pltpu.VMEM((BPW,), jnp.float32),
            pltpu.VMEM((BPW,), jnp.int32),
            pltpu.VMEM((BKT_PAD,), jnp.float32),
            pltpu.VMEM((CONST_PAD,), jnp.float32),
            pltpu.VMEM((TAIL_N * DIM,), jnp.float32),
            pltpu.VMEM((TS_ROWS_PAD * DIM,), jnp.float32),
        ] + [pltpu.VMEM((DIM, 128), jnp.float32)] * 8 + [
            pltpu.VMEM((OUT_PAD, BPW), jnp.float32),
            pltpu.SemaphoreType.DMA,
            pltpu.SemaphoreType.DMA,
        ],
    )(_sc_body)
    return f(user_id, timestamp, tabT, tail, tts, buckets_pad, consts)


def kernel(user_id, timestamp, user_table, ts_table, buckets, norm_mean,
           norm_var):
    n = buckets.shape[0]
    inv_step = (n - 1.0) / (buckets[-1] - buckets[0])
    denom = jnp.sqrt(norm_var + 1e-6)
    consts = jnp.concatenate([
        jnp.full((L,), inv_step, jnp.float32),
        jnp.full((L,), norm_mean, jnp.float32),
        jnp.full((L,), denom, jnp.float32),
        jnp.zeros((CONST_PAD - 3 * L,), jnp.float32),
    ])
    buckets_pad = jnp.concatenate(
        [buckets, jnp.full((BKT_PAD - n,), jnp.inf, jnp.float32)])
    tabT = user_table.T
    tail = user_table[TAIL_BASE:].reshape(-1)
    tts = jnp.pad(
        ts_table, ((0, TS_ROWS_PAD - ts_table.shape[0]), (0, 0))).reshape(-1)
    outp = _run(user_id, timestamp, tabT, tail, tts, buckets_pad, consts)
    return outp[:OUT_COLS].T
